# baseline (speedup 1.0000x reference)
import functools

import jax
import jax.numpy as jnp
from jax import lax
from jax.experimental import pallas as pl
from jax.experimental.pallas import tpu as pltpu
from jax.experimental.pallas import tpu_sc as plsc

B = 1024
C = 1024
D = 256
BLK_B = 512
NUM_BLOCKS = B // BLK_B
LOSS_SCALE = 1.25 / B


def _dist_argmin_body(x_ref, c_ref, idx_ref, loss_ref,
                      bcat_ref, cnh_ref, acc_ref):
    i = pl.program_id(0)

    @pl.when(i == 0)
    def _():
        c = c_ref[...]
        ch = c.astype(jnp.bfloat16)
        cl = (c - ch.astype(jnp.float32)).astype(jnp.bfloat16)
        cht = ch.T
        bcat_ref[0:D, :] = cht
        bcat_ref[D:2 * D, :] = cht
        bcat_ref[2 * D:, :] = cl.T
        cnh_ref[...] = 0.5 * jnp.sum(c * c, axis=1, keepdims=True).T
        acc_ref[0] = 0.0

    x = x_ref[...]
    xh = x.astype(jnp.bfloat16)
    xl = (x - xh.astype(jnp.float32)).astype(jnp.bfloat16)
    a = jnp.concatenate([xh, xl, xh], axis=1)
    g = jax.lax.dot(a, bcat_ref[...], preferred_element_type=jnp.float32)
    w = g - cnh_ref[...]
    maxval = jnp.max(w, axis=1, keepdims=True)
    iota = jax.lax.broadcasted_iota(jnp.int32, w.shape, 1)
    idx_ref[...] = jnp.min(jnp.where(w == maxval, iota, C), axis=1)

    xn = jnp.sum(x * x, axis=1, keepdims=True)
    acc_ref[0] += jnp.sum(xn - 2.0 * maxval)

    @pl.when(i == NUM_BLOCKS - 1)
    def _():
        loss_ref[...] = jnp.full((1,), acc_ref[0] * LOSS_SCALE)


_dist_argmin = pl.pallas_call(
    _dist_argmin_body,
    grid=(NUM_BLOCKS,),
    in_specs=[
        pl.BlockSpec((BLK_B, D), lambda i: (i, 0)),
        pl.BlockSpec((C, D), lambda i: (0, 0)),
    ],
    out_specs=[
        pl.BlockSpec((BLK_B,), lambda i: (i,)),
        pl.BlockSpec((1,), lambda i: (0,)),
    ],
    out_shape=[
        jax.ShapeDtypeStruct((B,), jnp.int32),
        jax.ShapeDtypeStruct((1,), jnp.float32),
    ],
    scratch_shapes=[
        pltpu.VMEM((3 * D, C), jnp.bfloat16),
        pltpu.VMEM((1, C), jnp.float32),
        pltpu.SMEM((1,), jnp.float32),
    ],
)


@functools.lru_cache(maxsize=1)
def _make_sc_gather():
    info = plsc.get_sparse_core_info()
    nc, ns = info.num_cores, info.num_subcores
    b_per_w = B // (nc * ns)

    @functools.partial(
        pl.kernel,
        mesh=plsc.VectorSubcoreMesh(core_axis_name="c", subcore_axis_name="s"),
        out_type=jax.ShapeDtypeStruct((B, D), jnp.float32),
        scratch_types=[
            pltpu.VMEM((b_per_w,), jnp.int32),
            pltpu.VMEM((b_per_w, D), jnp.float32),
            pltpu.SemaphoreType.DMA,
        ],
    )
    def _sc_gather(table_hbm, idx_hbm, out_hbm, idx_v, rows_v, sem):
        wid = lax.axis_index("s") * nc + lax.axis_index("c")
        base = wid * b_per_w
        pltpu.sync_copy(idx_hbm.at[pl.ds(base, b_per_w)], idx_v)
        pltpu.async_copy(table_hbm.at[idx_v], rows_v, sem).wait()
        pltpu.sync_copy(rows_v, out_hbm.at[pl.ds(base, b_per_w)])

    return _sc_gather


def kernel(x, codes):
    codes2d = codes[0]
    indices, loss1 = _dist_argmin(x, codes2d)
    quantized = _make_sc_gather()(codes2d, indices)
    return quantized, indices, loss1[0]


# speedup vs baseline: 1.0168x; 1.0168x over previous
import functools

import jax
import jax.numpy as jnp
from jax import lax
from jax.experimental import pallas as pl
from jax.experimental.pallas import tpu as pltpu
from jax.experimental.pallas import tpu_sc as plsc

B = 1024
C = 1024
D = 256
BLK_B = 1024
NUM_BLOCKS = B // BLK_B
LOSS_SCALE = 1.25 / B


def _dist_argmin_body(x_ref, c_ref, idx_ref, loss_ref,
                      bcat_ref, cnh_ref, acc_ref):
    i = pl.program_id(0)

    @pl.when(i == 0)
    def _():
        c = c_ref[...]
        ch = c.astype(jnp.bfloat16)
        cl = (c - ch.astype(jnp.float32)).astype(jnp.bfloat16)
        cht = ch.T
        bcat_ref[0:D, :] = cht
        bcat_ref[D:2 * D, :] = cht
        bcat_ref[2 * D:, :] = cl.T
        cnh_ref[...] = 0.5 * jnp.sum(c * c, axis=1, keepdims=True).T
        acc_ref[0] = 0.0

    x = x_ref[...]
    xh = x.astype(jnp.bfloat16)
    xl = (x - xh.astype(jnp.float32)).astype(jnp.bfloat16)
    a = jnp.concatenate([xh, xl, xh], axis=1)
    g = jax.lax.dot(a, bcat_ref[...], preferred_element_type=jnp.float32)
    w = g - cnh_ref[...]
    maxval = jnp.max(w, axis=1, keepdims=True)
    iota = jax.lax.broadcasted_iota(jnp.int32, w.shape, 1)
    idx_ref[...] = jnp.min(jnp.where(w == maxval, iota, C), axis=1)

    xn = jnp.sum(x * x, axis=1, keepdims=True)
    acc_ref[0] += jnp.sum(xn - 2.0 * maxval)

    @pl.when(i == NUM_BLOCKS - 1)
    def _():
        loss_ref[...] = jnp.full((1,), acc_ref[0] * LOSS_SCALE)


_dist_argmin = pl.pallas_call(
    _dist_argmin_body,
    grid=(NUM_BLOCKS,),
    in_specs=[
        pl.BlockSpec((BLK_B, D), lambda i: (i, 0)),
        pl.BlockSpec((C, D), lambda i: (0, 0)),
    ],
    out_specs=[
        pl.BlockSpec((BLK_B,), lambda i: (i,)),
        pl.BlockSpec((1,), lambda i: (0,)),
    ],
    out_shape=[
        jax.ShapeDtypeStruct((B,), jnp.int32),
        jax.ShapeDtypeStruct((1,), jnp.float32),
    ],
    scratch_shapes=[
        pltpu.VMEM((3 * D, C), jnp.bfloat16),
        pltpu.VMEM((1, C), jnp.float32),
        pltpu.SMEM((1,), jnp.float32),
    ],
)


@functools.lru_cache(maxsize=1)
def _make_sc_gather():
    info = plsc.get_sparse_core_info()
    nc, ns = info.num_cores, info.num_subcores
    b_per_w = B // (nc * ns)

    half = b_per_w // 2

    @functools.partial(
        pl.kernel,
        mesh=plsc.VectorSubcoreMesh(core_axis_name="c", subcore_axis_name="s"),
        out_type=jax.ShapeDtypeStruct((B, D), jnp.float32),
        scratch_types=[
            pltpu.VMEM((b_per_w,), jnp.int32),
            pltpu.VMEM((b_per_w, D), jnp.float32),
            pltpu.SemaphoreType.DMA,
            pltpu.SemaphoreType.DMA,
        ],
    )
    def _sc_gather(table_hbm, idx_hbm, out_hbm, idx_v, rows_v, sem0, sem1):
        wid = lax.axis_index("s") * nc + lax.axis_index("c")
        base = wid * b_per_w
        pltpu.sync_copy(idx_hbm.at[pl.ds(base, b_per_w)], idx_v)
        g0 = pltpu.async_copy(
            table_hbm.at[idx_v.at[pl.ds(0, half)]],
            rows_v.at[pl.ds(0, half)], sem0)
        g1 = pltpu.async_copy(
            table_hbm.at[idx_v.at[pl.ds(half, half)]],
            rows_v.at[pl.ds(half, half)], sem1)
        g0.wait()
        pltpu.sync_copy(rows_v.at[pl.ds(0, half)],
                        out_hbm.at[pl.ds(base, half)])
        g1.wait()
        pltpu.sync_copy(rows_v.at[pl.ds(half, half)],
                        out_hbm.at[pl.ds(base + half, half)])

    return _sc_gather


def kernel(x, codes):
    codes2d = codes[0]
    indices, loss1 = _dist_argmin(x, codes2d)
    quantized = _make_sc_gather()(codes2d, indices)
    return quantized, indices, loss1[0]


# speedup vs baseline: 1.0181x; 1.0013x over previous
import functools

import jax
import jax.numpy as jnp
from jax import lax
from jax.experimental import pallas as pl
from jax.experimental.pallas import tpu as pltpu
from jax.experimental.pallas import tpu_sc as plsc

B = 1024
C = 1024
D = 256
LOSS_SCALE = 1.25 / B


def _dist_argmin_body(x_ref, c_ref, idx_ref, loss_ref, bcat_ref, cnh_ref):
    c = c_ref[...]
    ch = c.astype(jnp.bfloat16)
    cl = (c - ch.astype(jnp.float32)).astype(jnp.bfloat16)
    cht = ch.T
    bcat_ref[0:D, :] = cht
    bcat_ref[D:2 * D, :] = cht
    bcat_ref[2 * D:, :] = cl.T
    cnh_ref[...] = 0.5 * jnp.sum(c * c, axis=1, keepdims=True).T

    x = x_ref[...]
    xh = x.astype(jnp.bfloat16)
    xl = (x - xh.astype(jnp.float32)).astype(jnp.bfloat16)
    a = jnp.concatenate([xh, xl, xh], axis=1)
    g = jax.lax.dot(a, bcat_ref[...], preferred_element_type=jnp.float32)
    w = g - cnh_ref[...]
    maxval = jnp.max(w, axis=1, keepdims=True)
    iota = jax.lax.broadcasted_iota(jnp.int32, w.shape, 1)
    idx_ref[...] = jnp.min(jnp.where(w == maxval, iota, C), axis=1)

    xn = jnp.sum(x * x, axis=1, keepdims=True)
    loss_ref[...] = jnp.full((1,), jnp.sum(xn - 2.0 * maxval) * LOSS_SCALE)


_dist_argmin = pl.pallas_call(
    _dist_argmin_body,
    out_shape=[
        jax.ShapeDtypeStruct((B,), jnp.int32),
        jax.ShapeDtypeStruct((1,), jnp.float32),
    ],
    scratch_shapes=[
        pltpu.VMEM((3 * D, C), jnp.bfloat16),
        pltpu.VMEM((1, C), jnp.float32),
    ],
)


@functools.lru_cache(maxsize=1)
def _make_sc_gather():
    info = plsc.get_sparse_core_info()
    nc, ns = info.num_cores, info.num_subcores
    b_per_w = B // (nc * ns)

    @functools.partial(
        pl.kernel,
        mesh=plsc.VectorSubcoreMesh(core_axis_name="c", subcore_axis_name="s"),
        out_type=jax.ShapeDtypeStruct((B, D), jnp.float32),
        scratch_types=[
            pltpu.VMEM((b_per_w,), jnp.int32),
            pltpu.VMEM((b_per_w, D), jnp.float32),
            pltpu.SemaphoreType.DMA,
        ],
    )
    def _sc_gather(table_hbm, idx_hbm, out_hbm, idx_v, rows_v, sem):
        wid = lax.axis_index("s") * nc + lax.axis_index("c")
        base = wid * b_per_w
        pltpu.sync_copy(idx_hbm.at[pl.ds(base, b_per_w)], idx_v)
        pltpu.async_copy(table_hbm.at[idx_v], rows_v, sem).wait()
        pltpu.sync_copy(rows_v, out_hbm.at[pl.ds(base, b_per_w)])

    return _sc_gather


def kernel(x, codes):
    codes2d = codes[0]
    indices, loss1 = _dist_argmin(x, codes2d)
    quantized = _make_sc_gather()(codes2d, indices)
    return quantized, indices, loss1[0]


# speedup vs baseline: 1.0592x; 1.0404x over previous
import functools

import jax
import jax.numpy as jnp
from jax import lax
from jax.experimental import pallas as pl
from jax.experimental.pallas import tpu as pltpu
from jax.experimental.pallas import tpu_sc as plsc

B = 1024
C = 1024
D = 256
LOSS_SCALE = 1.25 / B


def _dist_argmin_body(x_ref, c_ref, idx_ref, loss_ref, bcat_ref, cnh_ref):
    c = c_ref[...]
    ch = c.astype(jnp.bfloat16)
    cl = (c - ch.astype(jnp.float32)).astype(jnp.bfloat16)
    cht = ch.T
    bcat_ref[0:D, :] = cht
    bcat_ref[D:2 * D, :] = cht
    bcat_ref[2 * D:, :] = cl.T
    cnh_ref[...] = 0.5 * jnp.sum(c * c, axis=1, keepdims=True).T

    x = x_ref[...]
    xh = x.astype(jnp.bfloat16)
    xl = (x - xh.astype(jnp.float32)).astype(jnp.bfloat16)
    a = jnp.concatenate([xh, xl, xh], axis=1)
    g = jax.lax.dot(a, bcat_ref[...], preferred_element_type=jnp.float32)
    w = g - cnh_ref[...]
    maxval = jnp.max(w, axis=1, keepdims=True)
    iota = jax.lax.broadcasted_iota(jnp.int32, w.shape, 1)
    idx_ref[...] = jnp.min(jnp.where(w == maxval, iota, C), axis=1)

    xn = jnp.sum(x * x, axis=1, keepdims=True)
    loss_ref[...] = jnp.full((1,), jnp.sum(xn - 2.0 * maxval) * LOSS_SCALE)


_dist_argmin = pl.pallas_call(
    _dist_argmin_body,
    out_shape=[
        jax.ShapeDtypeStruct((B,), jnp.int32),
        jax.ShapeDtypeStruct((1,), jnp.float32),
    ],
    scratch_shapes=[
        pltpu.VMEM((3 * D, C), jnp.bfloat16),
        pltpu.VMEM((1, C), jnp.float32),
    ],
)


@functools.lru_cache(maxsize=1)
def _make_sc_gather():
    info = plsc.get_sparse_core_info()
    nc, ns = 1, info.num_subcores
    b_per_w = B // (nc * ns)

    @functools.partial(
        pl.kernel,
        mesh=plsc.VectorSubcoreMesh(core_axis_name="c", subcore_axis_name="s", num_cores=1),
        out_type=jax.ShapeDtypeStruct((B, D), jnp.float32),
        scratch_types=[
            pltpu.VMEM((b_per_w,), jnp.int32),
            pltpu.VMEM((b_per_w, D), jnp.float32),
            pltpu.SemaphoreType.DMA,
        ],
    )
    def _sc_gather(table_hbm, idx_hbm, out_hbm, idx_v, rows_v, sem):
        wid = lax.axis_index("s") * nc + lax.axis_index("c")
        base = wid * b_per_w
        pltpu.sync_copy(idx_hbm.at[pl.ds(base, b_per_w)], idx_v)
        pltpu.async_copy(table_hbm.at[idx_v], rows_v, sem).wait()
        pltpu.sync_copy(rows_v, out_hbm.at[pl.ds(base, b_per_w)])

    return _sc_gather


def kernel(x, codes):
    codes2d = codes[0]
    indices, loss1 = _dist_argmin(x, codes2d)
    quantized = _make_sc_gather()(codes2d, indices)
    return quantized, indices, loss1[0]
